# baseline (speedup 1.0000x reference)
import jax
import jax.numpy as jnp
from jax import lax
from jax.experimental import pallas as pl
from jax.experimental.pallas import tpu as pltpu

E = 8
K = 2
H = 1024
I = 1408
T = 1024
N = T * K
BM = 256
TILES_M = N // BM
NUM_TILES = TILES_M + E - 1

_DOT = (((1,), (1,)), ((), ()))


def _router_body(logits_t_ref, routes_ref, meta_ref):
    lt = logits_t_ref[...]
    mx = jnp.max(lt, axis=0, keepdims=True)
    ex = jnp.exp(lt - mx)
    probs = ex / jnp.sum(ex, axis=0, keepdims=True)

    idx = lax.broadcasted_iota(jnp.int32, (E, T), 0).astype(jnp.float32)
    l1 = jnp.max(lt, axis=0, keepdims=True)
    i1 = jnp.min(jnp.where(lt == l1, idx, 99.0), axis=0, keepdims=True)
    oh0 = (idx == i1).astype(jnp.float32)
    lmasked = jnp.where(oh0 > 0.0, -jnp.inf, lt)
    l2 = jnp.max(lmasked, axis=0, keepdims=True)
    i2 = jnp.min(jnp.where(lmasked == l2, idx, 99.0), axis=0, keepdims=True)
    oh1 = (idx == i2).astype(jnp.float32)
    m1 = jnp.sum(oh0 * probs, axis=0, keepdims=True)
    m2 = jnp.sum(oh1 * probs, axis=0, keepdims=True)
    s = m1 + m2
    w0 = m1 / s
    w1 = m2 / s

    rowsum = oh0 + oh1
    strict = (lax.broadcasted_iota(jnp.int32, (T, T), 0)
              < lax.broadcasted_iota(jnp.int32, (T, T), 1)).astype(jnp.float32)
    carry = lax.dot_general(rowsum, strict, (((1,), (0,)), ((), ())),
                            preferred_element_type=jnp.float32)
    tot = jnp.sum(rowsum, axis=1, keepdims=True)
    u8 = (lax.broadcasted_iota(jnp.int32, (E, E), 1)
          < lax.broadcasted_iota(jnp.int32, (E, E), 0)).astype(jnp.float32)
    off = lax.dot_general(u8, tot, (((1,), (0,)), ((), ())),
                          precision=lax.Precision.HIGHEST,
                          preferred_element_type=jnp.float32)
    posvec = carry + off
    pos0 = jnp.sum(oh0 * posvec, axis=0, keepdims=True)
    pos1 = jnp.sum(oh1 * posvec, axis=0, keepdims=True)
    routes_ref[...] = jnp.concatenate([pos0, pos1, w0, w1], axis=0)

    nonempty = tot > 0.0
    first_m = jnp.floor(off * (1.0 / BM))
    last_m = jnp.where(nonempty, jnp.floor((off + tot - 1.0) * (1.0 / BM)),
                       first_m - 1.0)
    ntiles = jnp.where(nonempty, last_m - first_m + 1.0, 0.0)
    starts = lax.dot_general(u8, ntiles, (((1,), (0,)), ((), ())),
                             precision=lax.Precision.HIGHEST,
                             preferred_element_type=jnp.float32)
    total = jnp.sum(ntiles, axis=0, keepdims=True)
    ones16 = jnp.ones((NUM_TILES + 1, 1), jnp.float32)

    def brow(col):
        return lax.dot_general(ones16, col, (((1,), (1,)), ((), ())),
                               precision=lax.Precision.HIGHEST,
                               preferred_element_type=jnp.float32)

    tt = lax.broadcasted_iota(jnp.int32, (NUM_TILES + 1, 1), 0).astype(jnp.float32)
    starts_inc_b = brow(starts + ntiles)
    g_ids = jnp.sum((tt >= starts_inc_b).astype(jnp.float32), axis=1,
                    keepdims=True)
    iota8c = lax.broadcasted_iota(jnp.int32, (E, 1), 0).astype(jnp.float32)
    g_last = jnp.max(jnp.where(nonempty, iota8c, -1.0), axis=0, keepdims=True)
    valid = tt < total
    g_ids = jnp.where(valid, jnp.minimum(g_ids, float(E - 1)), g_last)
    oh_g = (lax.broadcasted_iota(jnp.int32, (NUM_TILES + 1, E), 1)
            .astype(jnp.float32) == g_ids).astype(jnp.float32)

    def sel(col):
        return jnp.sum(oh_g * brow(col), axis=1, keepdims=True)

    m_ids = jnp.where(valid, sel(first_m) + tt - sel(starts),
                      float(TILES_M - 1))
    lo = jnp.where(valid, sel(off), 0.0)
    hi = jnp.where(valid, sel(off + tot), 0.0)
    meta_ref[...] = jnp.concatenate([g_ids, m_ids, lo, hi],
                                    axis=1).astype(jnp.int32)


def _router(router_logits):
    return pl.pallas_call(
        _router_body,
        out_shape=(
            jax.ShapeDtypeStruct((4, T), jnp.float32),
            jax.ShapeDtypeStruct((NUM_TILES + 1, 4), jnp.int32),
        ),
    )(router_logits.T)


def _moe_body(meta_ref, routes_ref, x_ref, w1_ref, w2_ref, out_ref):
    t = pl.program_id(0)
    m = meta_ref[t, 1]
    lo = meta_ref[t, 2]
    hi = meta_ref[t, 3]
    rr = m * BM + lax.broadcasted_iota(jnp.int32, (BM, 1), 0)
    rrf = rr.astype(jnp.float32)
    pos0 = routes_ref[0:1, :]
    pos1 = routes_ref[1:2, :]
    w0 = routes_ref[2:3, :]
    w1 = routes_ref[3:4, :]
    eq0 = (rrf == pos0).astype(jnp.float32)
    eq1 = (rrf == pos1).astype(jnp.float32)

    bf = jnp.bfloat16
    disp = (eq0 + eq1).astype(bf)
    xb = lax.dot_general(disp, x_ref[...].astype(bf),
                         (((1,), (0,)), ((), ())),
                         preferred_element_type=jnp.float32)
    gate_up = lax.dot_general(xb.astype(bf), w1_ref[0].astype(bf), _DOT,
                              preferred_element_type=jnp.float32)
    gate = gate_up[:, :I]
    up = gate_up[:, I:]
    act = gate * jax.nn.sigmoid(gate) * up
    y = lax.dot_general(act.astype(bf), w2_ref[0].astype(bf), _DOT,
                        preferred_element_type=jnp.float32)

    wrow = (lax.dot_general(eq0, w0, _DOT, precision=lax.Precision.HIGHEST,
                            preferred_element_type=jnp.float32)
            + lax.dot_general(eq1, w1, _DOT,
                              precision=lax.Precision.HIGHEST,
                              preferred_element_type=jnp.float32))
    maskf = ((rr >= lo) & (rr < hi)).astype(jnp.float32)
    ym = y * (wrow * maskf)

    prev_m = meta_ref[jnp.maximum(t - 1, 0), 1]
    first = (t == 0) | (m != prev_m)

    @pl.when(first)
    def _():
        out_ref[...] = ym

    @pl.when(jnp.logical_not(first))
    def _():
        out_ref[...] = out_ref[...] + ym


def _grouped_mlp(meta, routes, x, w1, w2):
    grid_spec = pltpu.PrefetchScalarGridSpec(
        num_scalar_prefetch=1,
        grid=(NUM_TILES,),
        in_specs=[
            pl.BlockSpec((4, T), lambda t, meta: (0, 0)),
            pl.BlockSpec((T, H), lambda t, meta: (0, 0)),
            pl.BlockSpec((1, 2 * I, H), lambda t, meta: (meta[t, 0], 0, 0)),
            pl.BlockSpec((1, H, I), lambda t, meta: (meta[t, 0], 0, 0)),
        ],
        out_specs=pl.BlockSpec((BM, H), lambda t, meta: (meta[t, 1], 0)),
    )
    return pl.pallas_call(
        _moe_body,
        grid_spec=grid_spec,
        out_shape=jax.ShapeDtypeStruct((N, H), jnp.float32),
        compiler_params=pltpu.CompilerParams(
            dimension_semantics=("arbitrary",)),
    )(meta, routes, x, w1, w2)


def _combine_body(routes_ref, ys_ref, out_ref):
    pos0 = routes_ref[0:1, :]
    pos1 = routes_ref[1:2, :]
    nn = lax.broadcasted_iota(jnp.int32, (N, T), 0).astype(jnp.float32)
    g0 = (nn == pos0).astype(jnp.bfloat16)
    g1 = (nn == pos1).astype(jnp.bfloat16)
    out_ref[...] = lax.dot_general(
        g0 + g1, ys_ref[...].astype(jnp.bfloat16),
        (((0,), (0,)), ((), ())),
        preferred_element_type=jnp.float32)


def _combine(routes, ys):
    return pl.pallas_call(
        _combine_body,
        out_shape=jax.ShapeDtypeStruct((T, H), jnp.float32),
    )(routes, ys)


def kernel(x, router_logits, w1, w2):
    routes, meta = _router(router_logits.astype(jnp.float32))
    ys = _grouped_mlp(meta, routes, x, w1, w2)
    out = _combine(routes, ys)
    return out.reshape(T, 1, H)


# speedup vs baseline: 1.0628x; 1.0628x over previous
import jax
import jax.numpy as jnp
from jax import lax
from jax.experimental import pallas as pl
from jax.experimental.pallas import tpu as pltpu

E = 8
K = 2
H = 1024
I = 1408
T = 1024
N = T * K
BM = 256
TILES_M = N // BM
NUM_TILES = TILES_M + E - 1

_DOT = (((1,), (1,)), ((), ()))


def _router_body(logits_t_ref, routes_ref, meta_ref):
    lt = logits_t_ref[...]
    mx = jnp.max(lt, axis=0, keepdims=True)
    ex = jnp.exp(lt - mx)
    probs = ex / jnp.sum(ex, axis=0, keepdims=True)

    idx = lax.broadcasted_iota(jnp.int32, (E, T), 0).astype(jnp.float32)
    l1 = jnp.max(lt, axis=0, keepdims=True)
    i1 = jnp.min(jnp.where(lt == l1, idx, 99.0), axis=0, keepdims=True)
    oh0 = (idx == i1).astype(jnp.float32)
    lmasked = jnp.where(oh0 > 0.0, -jnp.inf, lt)
    l2 = jnp.max(lmasked, axis=0, keepdims=True)
    i2 = jnp.min(jnp.where(lmasked == l2, idx, 99.0), axis=0, keepdims=True)
    oh1 = (idx == i2).astype(jnp.float32)
    m1 = jnp.sum(oh0 * probs, axis=0, keepdims=True)
    m2 = jnp.sum(oh1 * probs, axis=0, keepdims=True)
    s = m1 + m2
    w0 = m1 / s
    w1 = m2 / s

    rowsum = oh0 + oh1
    strict = (lax.broadcasted_iota(jnp.int32, (T, T), 0)
              < lax.broadcasted_iota(jnp.int32, (T, T), 1)).astype(jnp.float32)
    carry = lax.dot_general(rowsum, strict, (((1,), (0,)), ((), ())),
                            preferred_element_type=jnp.float32)
    tot = jnp.sum(rowsum, axis=1, keepdims=True)
    u8 = (lax.broadcasted_iota(jnp.int32, (E, E), 1)
          < lax.broadcasted_iota(jnp.int32, (E, E), 0)).astype(jnp.float32)
    off = lax.dot_general(u8, tot, (((1,), (0,)), ((), ())),
                          precision=lax.Precision.HIGHEST,
                          preferred_element_type=jnp.float32)
    posvec = carry + off
    pos0 = jnp.sum(oh0 * posvec, axis=0, keepdims=True)
    pos1 = jnp.sum(oh1 * posvec, axis=0, keepdims=True)
    routes_ref[...] = jnp.concatenate([pos0, pos1, w0, w1], axis=0)

    nonempty = tot > 0.0
    first_m = jnp.floor(off * (1.0 / BM))
    last_m = jnp.where(nonempty, jnp.floor((off + tot - 1.0) * (1.0 / BM)),
                       first_m - 1.0)
    ntiles = jnp.where(nonempty, last_m - first_m + 1.0, 0.0)
    starts = lax.dot_general(u8, ntiles, (((1,), (0,)), ((), ())),
                             precision=lax.Precision.HIGHEST,
                             preferred_element_type=jnp.float32)
    total = jnp.sum(ntiles, axis=0, keepdims=True)
    ones16 = jnp.ones((NUM_TILES + 1, 1), jnp.float32)

    def brow(col):
        return lax.dot_general(ones16, col, (((1,), (1,)), ((), ())),
                               precision=lax.Precision.HIGHEST,
                               preferred_element_type=jnp.float32)

    tt = lax.broadcasted_iota(jnp.int32, (NUM_TILES + 1, 1), 0).astype(jnp.float32)
    starts_inc_b = brow(starts + ntiles)
    g_ids = jnp.sum((tt >= starts_inc_b).astype(jnp.float32), axis=1,
                    keepdims=True)
    iota8c = lax.broadcasted_iota(jnp.int32, (E, 1), 0).astype(jnp.float32)
    g_last = jnp.max(jnp.where(nonempty, iota8c, -1.0), axis=0, keepdims=True)
    valid = tt < total
    g_ids = jnp.where(valid, jnp.minimum(g_ids, float(E - 1)), g_last)
    oh_g = (lax.broadcasted_iota(jnp.int32, (NUM_TILES + 1, E), 1)
            .astype(jnp.float32) == g_ids).astype(jnp.float32)

    def sel(col):
        return jnp.sum(oh_g * brow(col), axis=1, keepdims=True)

    m_ids = jnp.where(valid, sel(first_m) + tt - sel(starts),
                      float(TILES_M - 1))
    lo = jnp.where(valid, sel(off), 0.0)
    hi = jnp.where(valid, sel(off + tot), 0.0)
    meta_ref[...] = jnp.concatenate([g_ids, m_ids, lo, hi],
                                    axis=1).astype(jnp.int32)


def _router(router_logits):
    return pl.pallas_call(
        _router_body,
        out_shape=(
            jax.ShapeDtypeStruct((4, T), jnp.float32),
            jax.ShapeDtypeStruct((NUM_TILES + 1, 4), jnp.int32),
        ),
    )(router_logits.T)


def _moe_body(meta_ref, routes_ref, x_ref, w1_ref, w2_ref, out_ref):
    t = pl.program_id(0)
    m = meta_ref[t, 1]
    lo = meta_ref[t, 2]
    hi = meta_ref[t, 3]
    rr = m * BM + lax.broadcasted_iota(jnp.int32, (BM, 1), 0)
    rrf = rr.astype(jnp.float32)
    pos0 = routes_ref[0:1, :]
    pos1 = routes_ref[1:2, :]
    w0 = routes_ref[2:3, :]
    w1 = routes_ref[3:4, :]
    eq0 = (rrf == pos0).astype(jnp.float32)
    eq1 = (rrf == pos1).astype(jnp.float32)

    xb = lax.dot_general(eq0 + eq1, x_ref[...], (((1,), (0,)), ((), ())),
                         preferred_element_type=jnp.float32)
    gate_up = lax.dot_general(xb, w1_ref[0], _DOT,
                              preferred_element_type=jnp.float32)
    gate = gate_up[:, :I]
    up = gate_up[:, I:]
    act = gate * jax.nn.sigmoid(gate) * up
    y = lax.dot_general(act, w2_ref[0], _DOT,
                        preferred_element_type=jnp.float32)

    maskf = ((rr >= lo) & (rr < hi)).astype(jnp.float32)
    w_comb = (w0 * eq0 + w1 * eq1) * maskf
    contrib = lax.dot_general(w_comb, y, (((0,), (0,)), ((), ())),
                              preferred_element_type=jnp.float32)

    @pl.when(t == 0)
    def _():
        out_ref[...] = contrib

    @pl.when(t != 0)
    def _():
        out_ref[...] = out_ref[...] + contrib


def _grouped_mlp(meta, routes, x, w1, w2):
    grid_spec = pltpu.PrefetchScalarGridSpec(
        num_scalar_prefetch=1,
        grid=(NUM_TILES,),
        in_specs=[
            pl.BlockSpec((4, T), lambda t, meta: (0, 0)),
            pl.BlockSpec((T, H), lambda t, meta: (0, 0)),
            pl.BlockSpec((1, 2 * I, H), lambda t, meta: (meta[t, 0], 0, 0)),
            pl.BlockSpec((1, H, I), lambda t, meta: (meta[t, 0], 0, 0)),
        ],
        out_specs=pl.BlockSpec((T, H), lambda t, meta: (0, 0)),
    )
    return pl.pallas_call(
        _moe_body,
        grid_spec=grid_spec,
        out_shape=jax.ShapeDtypeStruct((T, H), jnp.float32),
        compiler_params=pltpu.CompilerParams(
            dimension_semantics=("arbitrary",)),
    )(meta, routes, x, w1, w2)


def kernel(x, router_logits, w1, w2):
    routes, meta = _router(router_logits.astype(jnp.float32))
    out = _grouped_mlp(meta, routes, x, w1, w2)
    return out.reshape(T, 1, H)
